# baseline (speedup 1.0000x reference)
import jax
import jax.numpy as jnp
from jax.experimental import pallas as pl
from jax.experimental.pallas import tpu as pltpu


def _fused_gcn_kernel(f_ori_ref, f_aug_ref, w_ref, b_ref, adj_ref,
                      out_ori_ref, out_aug_ref, supp_ref):
    d = w_ref.shape[1]
    i = pl.program_id(0)

    @pl.when(i == 0)
    def _():
        supp_ref[:, :d] = jnp.dot(f_ori_ref[:], w_ref[:],
                                  preferred_element_type=jnp.float32)
        supp_ref[:, d:] = jnp.dot(f_aug_ref[:], w_ref[:],
                                  preferred_element_type=jnp.float32)

    acc = jnp.dot(adj_ref[:].astype(jnp.bfloat16),
                  supp_ref[:].astype(jnp.bfloat16),
                  preferred_element_type=jnp.float32)
    out = jnp.maximum(acc + b_ref[:], 0.0)
    out_ori_ref[:] = out[:, :d]
    out_aug_ref[:] = out[:, d:]


def kernel(feature_ori, feature_aug, adjacency, weight, bias):
    n, d_in = feature_ori.shape
    d_out = weight.shape[1]
    bm = 200
    bias_cat = jnp.concatenate([bias, bias]).reshape(1, 2 * d_out)
    out_ori, out_aug = pl.pallas_call(
        _fused_gcn_kernel,
        grid=(n // bm,),
        in_specs=[
            pl.BlockSpec((n, d_in), lambda i: (0, 0)),
            pl.BlockSpec((n, d_in), lambda i: (0, 0)),
            pl.BlockSpec((d_in, d_out), lambda i: (0, 0)),
            pl.BlockSpec((1, 2 * d_out), lambda i: (0, 0)),
            pl.BlockSpec((bm, n), lambda i: (i, 0)),
        ],
        out_specs=[
            pl.BlockSpec((bm, d_out), lambda i: (i, 0)),
            pl.BlockSpec((bm, d_out), lambda i: (i, 0)),
        ],
        out_shape=[
            jax.ShapeDtypeStruct((n, d_out), jnp.float32),
            jax.ShapeDtypeStruct((n, d_out), jnp.float32),
        ],
        scratch_shapes=[pltpu.VMEM((n, 2 * d_out), jnp.float32)],
    )(feature_ori, feature_aug, weight, bias_cat, adjacency)
    return (out_ori, out_aug)


# speedup vs baseline: 1.0019x; 1.0019x over previous
import jax
import jax.numpy as jnp
from jax.experimental import pallas as pl
from jax.experimental.pallas import tpu as pltpu


def _fused_gcn_kernel(f_ori_ref, f_aug_ref, w_ref, b_ref, adj_lo_ref, adj_hi_ref,
                      out_ori_ref, out_aug_ref, supp_ref):
    d = w_ref.shape[1]
    i = pl.program_id(0)

    @pl.when(i == 0)
    def _():
        supp_ref[:, :d] = jnp.dot(f_ori_ref[:], w_ref[:],
                                  preferred_element_type=jnp.float32)
        supp_ref[:, d:] = jnp.dot(f_aug_ref[:], w_ref[:],
                                  preferred_element_type=jnp.float32)

    supp = supp_ref[:].astype(jnp.bfloat16)
    acc_lo = jnp.dot(adj_lo_ref[0].astype(jnp.bfloat16), supp,
                     preferred_element_type=jnp.float32)
    acc_hi = jnp.dot(adj_hi_ref[0].astype(jnp.bfloat16), supp,
                     preferred_element_type=jnp.float32)
    out_lo = jnp.maximum(acc_lo + b_ref[:], 0.0)
    out_hi = jnp.maximum(acc_hi + b_ref[:], 0.0)
    out_ori_ref[0] = out_lo[:, :d]
    out_ori_ref[1] = out_hi[:, :d]
    out_aug_ref[0] = out_lo[:, d:]
    out_aug_ref[1] = out_hi[:, d:]


def kernel(feature_ori, feature_aug, adjacency, weight, bias):
    n, d_in = feature_ori.shape
    d_out = weight.shape[1]
    bm = 200
    h = n // 2
    adj3 = adjacency.reshape(2, h, n)
    bias_cat = jnp.concatenate([bias, bias]).reshape(1, 2 * d_out)
    out_ori, out_aug = pl.pallas_call(
        _fused_gcn_kernel,
        grid=(h // bm,),
        in_specs=[
            pl.BlockSpec((n, d_in), lambda i: (0, 0)),
            pl.BlockSpec((n, d_in), lambda i: (0, 0)),
            pl.BlockSpec((d_in, d_out), lambda i: (0, 0)),
            pl.BlockSpec((1, 2 * d_out), lambda i: (0, 0)),
            pl.BlockSpec((1, bm, n), lambda i: (0, i, 0)),
            pl.BlockSpec((1, bm, n), lambda i: (1, i, 0)),
        ],
        out_specs=[
            pl.BlockSpec((2, bm, d_out), lambda i: (0, i, 0)),
            pl.BlockSpec((2, bm, d_out), lambda i: (0, i, 0)),
        ],
        out_shape=[
            jax.ShapeDtypeStruct((2, h, d_out), jnp.float32),
            jax.ShapeDtypeStruct((2, h, d_out), jnp.float32),
        ],
        scratch_shapes=[pltpu.VMEM((n, 2 * d_out), jnp.float32)],
    )(feature_ori, feature_aug, weight, bias_cat, adj3, adj3)
    return (out_ori.reshape(n, d_out), out_aug.reshape(n, d_out))


# speedup vs baseline: 1.0171x; 1.0152x over previous
import jax
import jax.numpy as jnp
from jax.experimental import pallas as pl
from jax.experimental.pallas import tpu as pltpu


def _fused_gcn_kernel(f_ori_ref, f_aug_ref, w_ref, b_ref, adj_ref,
                      out_ori_ref, out_aug_ref, supp_ref):
    d = w_ref.shape[1]
    i = pl.program_id(0)

    @pl.when(i == 0)
    def _():
        supp_ref[:, :d] = jnp.dot(f_ori_ref[:], w_ref[:],
                                  preferred_element_type=jnp.float32
                                  ).astype(jnp.bfloat16)
        supp_ref[:, d:] = jnp.dot(f_aug_ref[:], w_ref[:],
                                  preferred_element_type=jnp.float32
                                  ).astype(jnp.bfloat16)

    acc = jnp.dot(adj_ref[:].astype(jnp.bfloat16), supp_ref[:],
                  preferred_element_type=jnp.float32)
    out = jnp.maximum(acc + b_ref[:], 0.0)
    out_ori_ref[:] = out[:, :d]
    out_aug_ref[:] = out[:, d:]


def kernel(feature_ori, feature_aug, adjacency, weight, bias):
    n, d_in = feature_ori.shape
    d_out = weight.shape[1]
    bm = 400
    bias_cat = jnp.concatenate([bias, bias]).reshape(1, 2 * d_out)
    out_ori, out_aug = pl.pallas_call(
        _fused_gcn_kernel,
        grid=(n // bm,),
        in_specs=[
            pl.BlockSpec((n, d_in), lambda i: (0, 0)),
            pl.BlockSpec((n, d_in), lambda i: (0, 0)),
            pl.BlockSpec((d_in, d_out), lambda i: (0, 0)),
            pl.BlockSpec((1, 2 * d_out), lambda i: (0, 0)),
            pl.BlockSpec((bm, n), lambda i: (i, 0)),
        ],
        out_specs=[
            pl.BlockSpec((bm, d_out), lambda i: (i, 0)),
            pl.BlockSpec((bm, d_out), lambda i: (i, 0)),
        ],
        out_shape=[
            jax.ShapeDtypeStruct((n, d_out), jnp.float32),
            jax.ShapeDtypeStruct((n, d_out), jnp.float32),
        ],
        scratch_shapes=[pltpu.VMEM((n, 2 * d_out), jnp.bfloat16)],
    )(feature_ori, feature_aug, weight, bias_cat, adjacency)
    return (out_ori, out_aug)


# speedup vs baseline: 1.0216x; 1.0044x over previous
import jax
import jax.numpy as jnp
from jax.experimental import pallas as pl
from jax.experimental.pallas import tpu as pltpu


def _fused_gcn_kernel(f_ori_ref, f_aug_ref, w_ref, b_ref, adj_ref,
                      out_ori_ref, out_aug_ref, supp_ref):
    d = w_ref.shape[1]
    i = pl.program_id(0)

    @pl.when(i == 0)
    def _():
        supp_ref[:, :d] = jnp.dot(f_ori_ref[:], w_ref[:],
                                  preferred_element_type=jnp.float32)
        supp_ref[:, d:] = jnp.dot(f_aug_ref[:], w_ref[:],
                                  preferred_element_type=jnp.float32)

    acc = jnp.dot(adj_ref[:], supp_ref[:], preferred_element_type=jnp.float32)
    out = jnp.maximum(acc + b_ref[:], 0.0)
    out_ori_ref[:] = out[:, :d]
    out_aug_ref[:] = out[:, d:]


def kernel(feature_ori, feature_aug, adjacency, weight, bias):
    n, d_in = feature_ori.shape
    d_out = weight.shape[1]
    bm = 400
    bias_cat = jnp.concatenate([bias, bias]).reshape(1, 2 * d_out)
    out_ori, out_aug = pl.pallas_call(
        _fused_gcn_kernel,
        grid=(n // bm,),
        in_specs=[
            pl.BlockSpec((n, d_in), lambda i: (0, 0)),
            pl.BlockSpec((n, d_in), lambda i: (0, 0)),
            pl.BlockSpec((d_in, d_out), lambda i: (0, 0)),
            pl.BlockSpec((1, 2 * d_out), lambda i: (0, 0)),
            pl.BlockSpec((bm, n), lambda i: (i, 0)),
        ],
        out_specs=[
            pl.BlockSpec((bm, d_out), lambda i: (i, 0)),
            pl.BlockSpec((bm, d_out), lambda i: (i, 0)),
        ],
        out_shape=[
            jax.ShapeDtypeStruct((n, d_out), jnp.float32),
            jax.ShapeDtypeStruct((n, d_out), jnp.float32),
        ],
        scratch_shapes=[pltpu.VMEM((n, 2 * d_out), jnp.float32)],
    )(feature_ori, feature_aug, weight, bias_cat, adjacency)
    return (out_ori, out_aug)
